# Initial kernel scaffold; baseline (speedup 1.0000x reference)
import jax
import jax.numpy as jnp
from jax.experimental import pallas as pl


def kernel(input1, input2):
    raise NotImplementedError("write your pallas kernel here")


# speedup vs baseline: 1.4937x; 1.4937x over previous
import functools

import jax
import jax.numpy as jnp
from jax import lax
from jax.experimental import pallas as pl
from jax.experimental.pallas import tpu as pltpu
from jax.experimental.pallas import tpu_sc as plsc

_H = 224
_W = 224
_PH = _H + 2
_PWD = _W + 2
_PPB = _H * _W
_PIMG = _PH * _PWD
_NB = 4
_NC = 96
_ROWS = _NB * _PIMG
_PXW = _PPB // 8
_PX = 128
_NCHK = _PXW // _PX


def _pad_body(x_ref, o_ref):
    x = x_ref[0]
    rows = jnp.concatenate([x[:, :1], x, x[:, -1:]], axis=1)
    o_ref[0] = jnp.concatenate([rows[:, :, :1], rows, rows[:, :, -1:]], axis=2)


def _pad(input1):
    cb = 8
    return pl.pallas_call(
        _pad_body,
        grid=(_NB, _NC // cb),
        in_specs=[pl.BlockSpec((1, cb, _H, _W), lambda b, c: (b, c, 0, 0))],
        out_specs=pl.BlockSpec((1, cb, _PH, _PWD), lambda b, c: (b, c, 0, 0)),
        out_shape=jax.ShapeDtypeStruct((_NB, _NC, _PH, _PWD), jnp.float32),
    )(input1)


def _prep_body(d_ref, jall, wall):
    d = d_ref[...]
    dh = d[:, 0]
    dw = d[:, 1]
    y = lax.broadcasted_iota(jnp.int32, (_NB, _H, _W), 1).astype(jnp.float32)
    x = lax.broadcasted_iota(jnp.int32, (_NB, _H, _W), 2).astype(jnp.float32)
    h_up = dh + y + 1.0
    w_up = dw + x + 1.0
    hf = jnp.floor(h_up).astype(jnp.int32)
    wf = jnp.floor(w_up).astype(jnp.int32)
    hc = jnp.clip(hf + 1, 0, _PH - 1)
    wc = jnp.clip(wf + 1, 0, _PWD - 1)
    hf = jnp.clip(hf, 0, _PH - 1)
    wf = jnp.clip(wf, 0, _PWD - 1)
    b = lax.broadcasted_iota(jnp.int32, (_NB, _H, _W), 0) * _PIMG
    jall[:, 0] = b + hf * _PWD + wf
    jall[:, 1] = b + hf * _PWD + wc
    jall[:, 2] = b + hc * _PWD + wf
    jall[:, 3] = b + hc * _PWD + wc
    dhw = hc.astype(jnp.float32) - h_up
    dww = wc.astype(jnp.float32) - w_up
    wall[:, 0] = dhw * dww
    wall[:, 1] = dhw * (1.0 - dww)
    wall[:, 2] = (1.0 - dhw) * dww
    wall[:, 3] = (1.0 - dww) * (1.0 - dhw)


def _prepare(input2):
    return pl.pallas_call(
        _prep_body,
        out_shape=(jax.ShapeDtypeStruct((_NB, 4, _H, _W), jnp.int32),
                   jax.ShapeDtypeStruct((_NB, 4, _H, _W), jnp.float32)),
    )(input2)


def _splat(vec, i):
    idx = jnp.zeros((16,), jnp.int32) + i
    return lax.gather(
        vec, idx[:, None],
        lax.GatherDimensionNumbers(offset_dims=(), collapsed_slice_dims=(0,),
                                   start_index_map=(0,)),
        slice_sizes=(1,),
        mode=lax.GatherScatterMode.PROMISE_IN_BOUNDS)


def _blend(wv, t0, t1, t2, t3, ocm):
    def gblock(g, _):
        s = pl.multiple_of(g * 16, 16)
        w00v = wv[0, pl.ds(s, 16)]
        w10v = wv[1, pl.ds(s, 16)]
        w01v = wv[2, pl.ds(s, 16)]
        w11v = wv[3, pl.ds(s, 16)]

        def pstep(i, _):
            p = s + i
            s00 = _splat(w00v, i)
            s10 = _splat(w10v, i)
            s01 = _splat(w01v, i)
            s11 = _splat(w11v, i)
            for cg in range(_NC // 16):
                cs = cg * 16
                acc = (t0[p, pl.ds(cs, 16)] * s00
                       + t1[p, pl.ds(cs, 16)] * s10
                       + t2[p, pl.ds(cs, 16)] * s01
                       + t3[p, pl.ds(cs, 16)] * s11)
                ocm[p, pl.ds(cs, 16)] = acc
            return 0

        lax.fori_loop(0, 16, pstep, 0)
        return 0

    lax.fori_loop(0, _PX // 16, gblock, 0)


def _warp_body(table, jh, wh, out_hbm,
               jva, jvb, wva, wvb,
               ta0, ta1, ta2, ta3, tb0, tb1, tb2, tb3,
               ocma, ocmb,
               sja, swa, sjb, swb,
               sa0, sa1, sa2, sa3, sb0, sb1, sb2, sb3):
    wid = lax.axis_index("s") * 2 + lax.axis_index("c")
    b = wid // 8
    pb0 = (wid % 8) * _PXW

    def sb_start(c, jv, wv, sj, sw):
        p0 = pb0 + c * _PX
        cj = pltpu.async_copy(jh.at[b, :, pl.ds(p0, _PX)], jv, sj)
        cw = pltpu.async_copy(wh.at[b, :, pl.ds(p0, _PX)], wv, sw)
        return cj, cw

    def gathers(jv, ts, sems):
        return [pltpu.async_copy(table.at[jv.at[t]], ts[t], sems[t])
                for t in range(4)]

    def out_copy(c, ocm):
        p0 = pb0 + c * _PX
        pltpu.sync_copy(ocm, out_hbm.at[b, pl.ds(p0, _PX), :])

    cj, cw = sb_start(0, jva, wva, sja, swa)
    cj.wait()
    cw.wait()

    def body(kk, _):
        ca = 2 * kk
        ga = gathers(jva, (ta0, ta1, ta2, ta3), (sa0, sa1, sa2, sa3))
        cjb, cwb = sb_start(ca + 1, jvb, wvb, sjb, swb)
        cjb.wait()
        cwb.wait()
        gb = gathers(jvb, (tb0, tb1, tb2, tb3), (sb0, sb1, sb2, sb3))
        for g in ga:
            g.wait()
        _blend(wva, ta0, ta1, ta2, ta3, ocma)
        out_copy(ca, ocma)
        cja, cwa = sb_start(ca + 2, jva, wva, sja, swa)
        for g in gb:
            g.wait()
        _blend(wvb, tb0, tb1, tb2, tb3, ocmb)
        out_copy(ca + 1, ocmb)
        cja.wait()
        cwa.wait()
        return 0

    lax.fori_loop(0, (_NCHK - 1) // 2, body, 0)

    ga = gathers(jva, (ta0, ta1, ta2, ta3), (sa0, sa1, sa2, sa3))
    for g in ga:
        g.wait()
    _blend(wva, ta0, ta1, ta2, ta3, ocma)
    out_copy(_NCHK - 1, ocma)


def _warp_sc(table, jall, wall):
    mesh = plsc.VectorSubcoreMesh(core_axis_name="c", subcore_axis_name="s")
    tap = pltpu.VMEM((_PX, _NC), jnp.float32)
    f = functools.partial(
        pl.kernel,
        mesh=mesh,
        compiler_params=pltpu.CompilerParams(use_tc_tiling_on_sc=False),
        out_type=jax.ShapeDtypeStruct((_NB, _PPB, _NC), jnp.float32),
        scratch_types=[
            pltpu.VMEM((4, _PX), jnp.int32),
            pltpu.VMEM((4, _PX), jnp.int32),
            pltpu.VMEM((4, _PX), jnp.float32),
            pltpu.VMEM((4, _PX), jnp.float32),
            tap, tap, tap, tap, tap, tap, tap, tap,
            pltpu.VMEM((_PX, _NC), jnp.float32),
            pltpu.VMEM((_PX, _NC), jnp.float32),
        ] + [pltpu.SemaphoreType.DMA] * 12,
    )(_warp_body)
    return f(table, jall, wall)


_TT = 896


def _tr_body(x_ref, o_ref):
    o_ref[0] = jnp.swapaxes(x_ref[0], 0, 1)


def _to_channel_major(out_pm):
    return pl.pallas_call(
        _tr_body,
        grid=(_NB, _PPB // _TT),
        in_specs=[pl.BlockSpec((1, _TT, _NC), lambda b, t: (b, t, 0))],
        out_specs=pl.BlockSpec((1, _NC, _TT), lambda b, t: (b, 0, t)),
        out_shape=jax.ShapeDtypeStruct((_NB, _NC, _PPB), jnp.float32),
    )(out_pm)


def kernel(input1, input2):
    padbuf = _pad(input1)
    table = padbuf.reshape(_ROWS, _NC)
    jall, wall = _prepare(input2)
    out_pm = _warp_sc(table,
                      jall.reshape(_NB, 4, _PPB),
                      wall.reshape(_NB, 4, _PPB))
    return _to_channel_major(out_pm).reshape(_NB, _NC, _H, _W)
